# baseline (speedup 1.0000x reference)
import functools

import jax
import jax.numpy as jnp
from jax import lax
from jax.experimental import pallas as pl
from jax.experimental.pallas import tpu as pltpu
from jax.experimental.pallas import tpu_sc as plsc

NUM_ITEMS = 100000
EMBED_DIM = 64
BATCH = 4096
HIST = 50

NUM_WORKERS = 32
BBLK = BATCH // NUM_WORKERS
LANES = 16
RB = BBLK // LANES
EB = EMBED_DIM // LANES
RB_E = EMBED_DIM // 8
LAG = 4


def _transpose_block(gbuf, tbuf, lane, diag):
    for eb in range(EB):
        cols = [diag[k] + eb * LANES for k in range(LANES)]

        @plsc.parallel_loop(0, RB, 1, unroll=2)
        def _rb_loop(rb):
            rows = lane + rb * LANES
            vals = {}
            for k in range(LANES):
                vals[k] = plsc.load_gather(gbuf, [rows, cols[k]])
                if k >= LAG:
                    plsc.store_scatter(tbuf, [cols[k - LAG], rows],
                                       vals.pop(k - LAG))
            for k in range(LANES - LAG, LANES):
                plsc.store_scatter(tbuf, [cols[k], rows], vals.pop(k))


def _gather_kernel(idx_hbm, table_hbm, out_hbm,
                   idx_v, gbuf0, gbuf1, gbuf2, tbuf0, tbuf1,
                   gsem0, gsem1, gsem2, wsem0, wsem1):
    c = lax.axis_index("c")
    s = lax.axis_index("s")
    wid = s * 2 + c
    bbase = wid * BBLK

    lane = lax.iota(jnp.int32, LANES)
    diag = [(lane + k) & (LANES - 1) for k in range(LANES)]

    pltpu.sync_copy(idx_hbm.at[wid], idx_v)

    def fire(h, gbuf, sem):
        pltpu.async_copy(table_hbm.at[idx_v.at[h]], gbuf, sem)

    def drain_gather(gbuf, sem):
        pltpu.make_async_copy(table_hbm.at[pl.ds(0, BBLK)], gbuf, sem).wait()

    def wb(h, tbuf, sem):
        for er in range(RB_E):
            pltpu.async_copy(tbuf.at[pl.ds(er * 8, 8)],
                             out_hbm.at[h * RB_E + er, wid], sem)

    def drain_wb(tbuf, sem):
        for er in range(RB_E):
            pltpu.make_async_copy(tbuf.at[pl.ds(er * 8, 8)],
                                  out_hbm.at[er, wid], sem).wait()

    gbufs = (gbuf0, gbuf1, gbuf2)
    gsems = (gsem0, gsem1, gsem2)
    tbufs = (tbuf0, tbuf1)
    wsems = (wsem0, wsem1)

    fire(0, gbuf0, gsem0)
    fire(1, gbuf1, gsem1)

    STEP = 6
    MAIN = (HIST - 2) // STEP

    def six_body(q, carry):
        h0 = STEP * q
        for j in range(STEP):
            h = h0 + j
            gi, ti = j % 3, j % 2
            drain_gather(gbufs[gi], gsems[gi])
            fire(h + 2, gbufs[(j + 2) % 3], gsems[(j + 2) % 3])
            if j >= 2:
                drain_wb(tbufs[ti], wsems[ti])
            else:
                @pl.when(q > 0)
                def _():
                    drain_wb(tbufs[ti], wsems[ti])
            _transpose_block(gbufs[gi], tbufs[ti], lane, diag)
            wb(h, tbufs[ti], wsems[ti])
        return carry

    lax.fori_loop(0, MAIN, six_body, None)

    for j in range(2):
        h = HIST - 2 + j
        drain_gather(gbufs[j], gsems[j])
        drain_wb(tbufs[j], wsems[j])
        _transpose_block(gbufs[j], tbufs[j], lane, diag)
        wb(h, tbufs[j], wsems[j])
    drain_wb(tbuf0, wsem0)
    drain_wb(tbuf1, wsem1)


@functools.partial(jax.jit, static_argnames=())
def kernel(item_inputs, item_embedding):
    idx = (item_inputs.astype(jnp.int32)
           .reshape(NUM_WORKERS, BBLK, HIST)
           .transpose(0, 2, 1)) * 2
    table2 = jnp.pad(item_embedding, ((0, 0), (0, EMBED_DIM))) \
        .reshape(2 * NUM_ITEMS, EMBED_DIM)
    mesh = plsc.VectorSubcoreMesh(core_axis_name="c", subcore_axis_name="s")
    out4 = pl.kernel(
        _gather_kernel,
        out_type=jax.ShapeDtypeStruct(
            (HIST * RB_E, NUM_WORKERS, 8, 128), jnp.float32),
        mesh=mesh,
        scratch_types=[
            pltpu.VMEM((HIST, BBLK), jnp.int32),
            pltpu.VMEM((BBLK, EMBED_DIM), jnp.float32),
            pltpu.VMEM((BBLK, EMBED_DIM), jnp.float32),
            pltpu.VMEM((BBLK, EMBED_DIM), jnp.float32),
            pltpu.VMEM((EMBED_DIM, BBLK), jnp.float32),
            pltpu.VMEM((EMBED_DIM, BBLK), jnp.float32),
            pltpu.SemaphoreType.DMA,
            pltpu.SemaphoreType.DMA,
            pltpu.SemaphoreType.DMA,
            pltpu.SemaphoreType.DMA,
            pltpu.SemaphoreType.DMA,
        ],
        compiler_params=pltpu.CompilerParams(
            use_tc_tiling_on_sc=False, needs_layout_passes=False),
    )(idx, table2)
    out = (out4.reshape(HIST, RB_E, NUM_WORKERS, 8, 128)
           .transpose(2, 4, 0, 1, 3)
           .reshape(BATCH, HIST, EMBED_DIM))
    return out


# speedup vs baseline: 1.0467x; 1.0467x over previous
import functools

import jax
import jax.numpy as jnp
from jax import lax
from jax.experimental import pallas as pl
from jax.experimental.pallas import tpu as pltpu
from jax.experimental.pallas import tpu_sc as plsc

NUM_ITEMS = 100000
EMBED_DIM = 64
BATCH = 4096
HIST = 50

NUM_WORKERS = 32
BBLK = BATCH // NUM_WORKERS
LANES = 16
RB = BBLK // LANES
EB = EMBED_DIM // LANES
RB_E = EMBED_DIM // 8
HALF = EMBED_DIM // 2
LAG = 4


def _transpose_block(gbuf, tbuf, lane, diag):
    for eb in range(EB):
        cols = [diag[k] + eb * LANES for k in range(LANES)]
        his = [c >> 5 for c in cols]
        los = [c & (HALF - 1) for c in cols]

        @plsc.parallel_loop(0, RB, 1, unroll=2)
        def _rb_loop(rb):
            rows = lane + rb * LANES
            rows2 = rows * 2
            vals = {}
            for k in range(LANES):
                vals[k] = plsc.load_gather(gbuf, [rows2 + his[k], los[k]])
                if k >= LAG:
                    plsc.store_scatter(tbuf, [cols[k - LAG], rows],
                                       vals.pop(k - LAG))
            for k in range(LANES - LAG, LANES):
                plsc.store_scatter(tbuf, [cols[k], rows], vals.pop(k))


def _gather_kernel(idx_hbm, table_hbm, out_hbm,
                   idx_v, gbuf0, gbuf1, tbuf0, tbuf1,
                   gsem0, gsem1, wsem0, wsem1):
    c = lax.axis_index("c")
    s = lax.axis_index("s")
    wid = s * 2 + c
    bbase = wid * BBLK

    lane = lax.iota(jnp.int32, LANES)
    diag = [(lane + k) & (LANES - 1) for k in range(LANES)]

    pltpu.sync_copy(idx_hbm.at[wid], idx_v)

    def fire(h, gbuf, sem):
        pltpu.async_copy(table_hbm.at[idx_v.at[2 * h]],
                         gbuf.at[pl.ds(0, BBLK)], sem)
        pltpu.async_copy(table_hbm.at[idx_v.at[2 * h + 1]],
                         gbuf.at[pl.ds(BBLK, BBLK)], sem)

    def drain_gather(gbuf, sem):
        for half in range(2):
            pltpu.make_async_copy(table_hbm.at[pl.ds(0, BBLK)],
                                  gbuf.at[pl.ds(half * BBLK, BBLK)],
                                  sem).wait()

    def wb(h, tbuf, sem):
        for er in range(RB_E):
            pltpu.async_copy(tbuf.at[pl.ds(er * 8, 8)],
                             out_hbm.at[h * RB_E + er, wid], sem)

    def drain_wb(tbuf, sem):
        for er in range(RB_E):
            pltpu.make_async_copy(tbuf.at[pl.ds(er * 8, 8)],
                                  out_hbm.at[er, wid], sem).wait()

    fire(0, gbuf0, gsem0)
    fire(1, gbuf1, gsem1)

    def pair_body(p, carry):
        h0 = 2 * p

        drain_gather(gbuf0, gsem0)

        @pl.when(p > 0)
        def _():
            drain_wb(tbuf0, wsem0)

        _transpose_block(gbuf0, tbuf0, lane, diag)

        @pl.when(p < HIST // 2 - 1)
        def _():
            fire(h0 + 2, gbuf0, gsem0)

        wb(h0, tbuf0, wsem0)

        drain_gather(gbuf1, gsem1)

        @pl.when(p > 0)
        def _():
            drain_wb(tbuf1, wsem1)

        _transpose_block(gbuf1, tbuf1, lane, diag)

        @pl.when(p < HIST // 2 - 1)
        def _():
            fire(h0 + 3, gbuf1, gsem1)

        wb(h0 + 1, tbuf1, wsem1)
        return carry

    lax.fori_loop(0, HIST // 2, pair_body, None)
    drain_wb(tbuf0, wsem0)
    drain_wb(tbuf1, wsem1)


@functools.partial(jax.jit, static_argnames=())
def kernel(item_inputs, item_embedding):
    idxw = (item_inputs.astype(jnp.int32)
            .reshape(NUM_WORKERS, BBLK, HIST)
            .transpose(0, 2, 1))
    idx = jnp.stack([idxw * 2, idxw * 2 + 1], axis=-1) \
        .reshape(NUM_WORKERS, 2 * HIST, BBLK)
    table2 = item_embedding.reshape(2 * NUM_ITEMS, HALF)
    mesh = plsc.VectorSubcoreMesh(core_axis_name="c", subcore_axis_name="s")
    out4 = pl.kernel(
        _gather_kernel,
        out_type=jax.ShapeDtypeStruct(
            (HIST * RB_E, NUM_WORKERS, 8, 128), jnp.float32),
        mesh=mesh,
        scratch_types=[
            pltpu.VMEM((2 * HIST, BBLK), jnp.int32),
            pltpu.VMEM((2 * BBLK, HALF), jnp.float32),
            pltpu.VMEM((2 * BBLK, HALF), jnp.float32),
            pltpu.VMEM((EMBED_DIM, BBLK), jnp.float32),
            pltpu.VMEM((EMBED_DIM, BBLK), jnp.float32),
            pltpu.SemaphoreType.DMA,
            pltpu.SemaphoreType.DMA,
            pltpu.SemaphoreType.DMA,
            pltpu.SemaphoreType.DMA,
        ],
        compiler_params=pltpu.CompilerParams(
            use_tc_tiling_on_sc=False, needs_layout_passes=False),
    )(idx, table2)
    out = (out4.reshape(HIST, RB_E, NUM_WORKERS, 8, 128)
           .transpose(2, 4, 0, 1, 3)
           .reshape(BATCH, HIST, EMBED_DIM))
    return out


# speedup vs baseline: 1.1379x; 1.0872x over previous
import functools

import jax
import jax.numpy as jnp
from jax import lax
from jax.experimental import pallas as pl
from jax.experimental.pallas import tpu as pltpu
from jax.experimental.pallas import tpu_sc as plsc

NUM_ITEMS = 100000
EMBED_DIM = 64
BATCH = 4096
HIST = 50

NUM_WORKERS = 32
BBLK = BATCH // NUM_WORKERS
LANES = 16
RB = BBLK // LANES
EB = EMBED_DIM // LANES
RB_E = EMBED_DIM // 8
LAG = 4


def _transpose_block(gbuf, tbuf, lane, diag):
    for eb in range(EB):
        cols = [diag[k] + eb * LANES for k in range(LANES)]

        @plsc.parallel_loop(0, RB, 1, unroll=2)
        def _rb_loop(rb):
            rows = lane + rb * LANES
            vals = {}
            for k in range(LANES):
                vals[k] = plsc.load_gather(gbuf, [rows, cols[k]])
                if k >= LAG:
                    plsc.store_scatter(tbuf, [cols[k - LAG], rows],
                                       vals.pop(k - LAG))
            for k in range(LANES - LAG, LANES):
                plsc.store_scatter(tbuf, [cols[k], rows], vals.pop(k))


def _gather_kernel(idx_hbm, table_hbm, out_hbm,
                   idx_v, gbuf0, gbuf1, tbuf0, tbuf1,
                   gsem0, gsem1, wsem0, wsem1):
    c = lax.axis_index("c")
    s = lax.axis_index("s")
    wid = s * 2 + c
    bbase = wid * BBLK

    lane = lax.iota(jnp.int32, LANES)
    diag = [(lane + k) & (LANES - 1) for k in range(LANES)]

    pltpu.sync_copy(idx_hbm.at[wid], idx_v)

    def fire(h, gbuf, sem):
        pltpu.async_copy(table_hbm.at[idx_v.at[h]], gbuf, sem)

    def drain_gather(gbuf, sem):
        pltpu.make_async_copy(table_hbm.at[pl.ds(0, BBLK)], gbuf, sem).wait()

    def wb(h, tbuf, sem):
        for er in range(RB_E):
            pltpu.async_copy(tbuf.at[pl.ds(er * 8, 8)],
                             out_hbm.at[h * RB_E + er, wid], sem)

    def drain_wb(tbuf, sem):
        for er in range(RB_E):
            pltpu.make_async_copy(tbuf.at[pl.ds(er * 8, 8)],
                                  out_hbm.at[er, wid], sem).wait()

    fire(0, gbuf0, gsem0)
    fire(1, gbuf1, gsem1)

    def pair_body(p, carry):
        h0 = 2 * p

        drain_gather(gbuf0, gsem0)

        @pl.when(p > 0)
        def _():
            drain_wb(tbuf0, wsem0)

        _transpose_block(gbuf0, tbuf0, lane, diag)

        @pl.when(p < HIST // 2 - 1)
        def _():
            fire(h0 + 2, gbuf0, gsem0)

        wb(h0, tbuf0, wsem0)

        drain_gather(gbuf1, gsem1)

        @pl.when(p > 0)
        def _():
            drain_wb(tbuf1, wsem1)

        _transpose_block(gbuf1, tbuf1, lane, diag)

        @pl.when(p < HIST // 2 - 1)
        def _():
            fire(h0 + 3, gbuf1, gsem1)

        wb(h0 + 1, tbuf1, wsem1)
        return carry

    lax.fori_loop(0, HIST // 2, pair_body, None)
    drain_wb(tbuf0, wsem0)
    drain_wb(tbuf1, wsem1)


@functools.partial(jax.jit, static_argnames=())
def kernel(item_inputs, item_embedding):
    idx = (item_inputs.astype(jnp.int32)
           .reshape(NUM_WORKERS, BBLK, HIST)
           .transpose(0, 2, 1)) * 2
    table2 = jnp.pad(item_embedding, ((0, 0), (0, EMBED_DIM))) \
        .reshape(2 * NUM_ITEMS, EMBED_DIM)
    mesh = plsc.VectorSubcoreMesh(core_axis_name="c", subcore_axis_name="s")
    out4 = pl.kernel(
        _gather_kernel,
        out_type=jax.ShapeDtypeStruct(
            (HIST * RB_E, NUM_WORKERS, 8, 128), jnp.float32),
        mesh=mesh,
        scratch_types=[
            pltpu.VMEM((HIST, BBLK), jnp.int32),
            pltpu.VMEM((BBLK, EMBED_DIM), jnp.float32),
            pltpu.VMEM((BBLK, EMBED_DIM), jnp.float32),
            pltpu.VMEM((EMBED_DIM, BBLK), jnp.float32),
            pltpu.VMEM((EMBED_DIM, BBLK), jnp.float32),
            pltpu.SemaphoreType.DMA,
            pltpu.SemaphoreType.DMA,
            pltpu.SemaphoreType.DMA,
            pltpu.SemaphoreType.DMA,
        ],
        compiler_params=pltpu.CompilerParams(
            use_tc_tiling_on_sc=False, needs_layout_passes=False),
    )(idx, table2)
    out = (out4.reshape(HIST, RB_E, NUM_WORKERS, 8, 128)
           .transpose(2, 4, 0, 1, 3)
           .reshape(BATCH, HIST, EMBED_DIM))
    return out
